# baseline (speedup 1.0000x reference)
import functools

import jax
import jax.numpy as jnp
from jax import lax
from jax.experimental import pallas as pl
from jax.experimental.pallas import tpu as pltpu
from jax.experimental.pallas import tpu_sc as plsc

N = 10000
R = 16
NB = 12
D = 128
E = 320000
NSEG = N * R
NTILES = 32
EPT = E // NTILES
CH = 80
NCH = EPT // CH
NPAD = 10240
RPT = NPAD // 16
BROWS = 64
SPT = NSEG // 16
EPAD = 327680
EPTP = EPAD // NTILES
CHE = 32
NCHE = EPTP // CHE
NSEGP = NPAD * R
SPTP = NSEGP // 16


def _keys_body(src_ref, dst_ref, ty_ref, ksrc_ref, kdst_ref):
    ksrc_ref[...] = ty_ref[...] * N + src_ref[...]
    kdst_ref[...] = dst_ref[...] * R + ty_ref[...]


def _make_keys(src, dst, ty):
    s2 = src.reshape(2500, 128)
    d2 = dst.reshape(2500, 128)
    t2 = ty.reshape(2500, 128)
    ks, kd = pl.pallas_call(
        _keys_body,
        out_shape=(jax.ShapeDtypeStruct((2500, 128), jnp.int32),
                   jax.ShapeDtypeStruct((2500, 128), jnp.int32)),
    )(s2, d2, t2)
    return ks.reshape(E), kd.reshape(E)


def _wrel_body(comp_ref, basis_ref, out_ref):
    r = pl.program_id(0)
    acc = comp_ref[r, 0] * basis_ref[0]
    for b in range(1, NB):
        acc = acc + comp_ref[r, b] * basis_ref[b]
    out_ref[...] = acc


def _make_wrel(basis, comp):
    wrel = pl.pallas_call(
        _wrel_body,
        grid=(R,),
        in_specs=[
            pl.BlockSpec(memory_space=pltpu.SMEM),
            pl.BlockSpec((NB, D, D), lambda r: (0, 0, 0)),
        ],
        out_specs=pl.BlockSpec((D, D), lambda r: (0, r)),
        out_shape=jax.ShapeDtypeStruct((D, R * D), jnp.float32),
    )(comp, basis)
    return wrel


def _mm_body(x_ref, w_ref, xt_ref):
    xt_ref[...] = jnp.dot(x_ref[...], w_ref[...],
                          preferred_element_type=jnp.float32)


def _matmul(x, wrel):
    return pl.pallas_call(
        _mm_body,
        grid=(R, 10),
        in_specs=[
            pl.BlockSpec((1000, D), lambda r, i: (i, 0)),
            pl.BlockSpec((D, D), lambda r, i: (0, r)),
        ],
        out_specs=pl.BlockSpec((1000, D), lambda r, i: (r * 10 + i, 0)),
        out_shape=jax.ShapeDtypeStruct((NSEG, D), jnp.float32),
    )(x, wrel)


def _inv_body(c0_ref, c1_ref, out_ref):
    c = c0_ref[...] + c1_ref[...]
    out_ref[...] = 1.0 / jnp.maximum(c, 1.0)


def _make_invw(cparts):
    c0 = cparts[0].reshape(1250, 128)
    c1 = cparts[1].reshape(1250, 128)
    inv = pl.pallas_call(
        _inv_body,
        out_shape=jax.ShapeDtypeStruct((1250, 128), jnp.float32),
    )(c0, c1)
    return inv.reshape(NSEG)


def _comb_body(p0_ref, p1_ref, x_ref, root_ref, b_ref, out_ref):
    rp = jnp.dot(x_ref[...], root_ref[...], preferred_element_type=jnp.float32)
    s = p0_ref[...] + p1_ref[...] + rp + b_ref[...]
    out_ref[...] = jnp.maximum(s, 0.0)


def _combine(p0, p1, x, root, bias):
    return pl.pallas_call(
        _comb_body,
        grid=(10,),
        in_specs=[
            pl.BlockSpec((1000, D), lambda i: (i, 0)),
            pl.BlockSpec((1000, D), lambda i: (i, 0)),
            pl.BlockSpec((1000, D), lambda i: (i, 0)),
            pl.BlockSpec((D, D), lambda i: (0, 0)),
            pl.BlockSpec((1, D), lambda i: (0, 0)),
        ],
        out_specs=pl.BlockSpec((1000, D), lambda i: (i, 0)),
        out_shape=jax.ShapeDtypeStruct((N, D), jnp.float32),
    )(p0, p1, x, root, bias.reshape(1, D))


def _final_body(p0_ref, p1_ref, x1_ref, emb_ref, root_ref, b_ref, out_ref):
    rp = jnp.dot(x1_ref[...], root_ref[...], preferred_element_type=jnp.float32)
    x2 = jnp.maximum(p0_ref[...] + p1_ref[...] + rp + b_ref[...], 0.0)
    out_ref[...] = jnp.concatenate((x2, x1_ref[...], emb_ref[...]), axis=1)


def _final(p0, p1, x1, emb, root, bias):
    return pl.pallas_call(
        _final_body,
        grid=(10,),
        in_specs=[
            pl.BlockSpec((1000, D), lambda i: (i, 0)),
            pl.BlockSpec((1000, D), lambda i: (i, 0)),
            pl.BlockSpec((1000, D), lambda i: (i, 0)),
            pl.BlockSpec((1000, D), lambda i: (i, 0)),
            pl.BlockSpec((D, D), lambda i: (0, 0)),
            pl.BlockSpec((1, D), lambda i: (0, 0)),
        ],
        out_specs=pl.BlockSpec((1000, 3 * D), lambda i: (i, 0)),
        out_shape=jax.ShapeDtypeStruct((N, 3 * D), jnp.float32),
    )(p0, p1, x1, emb, root, bias.reshape(1, D))


_MESH = plsc.VectorSubcoreMesh(core_axis_name="c", subcore_axis_name="s")


CRING = 5


@functools.partial(
    pl.kernel,
    out_type=jax.ShapeDtypeStruct((2 * NSEG,), jnp.float32),
    mesh=_MESH,
    scratch_types=[
        pltpu.VMEM((EPT,), jnp.int32),
        pltpu.VMEM((NCH, 1, CH), jnp.int32),
        pltpu.VMEM((CH,), jnp.float32),
        pltpu.VMEM((SPT,), jnp.float32),
        pltpu.VMEM_SHARED((NSEG,), jnp.float32),
        pltpu.SemaphoreType.DMA,
        pltpu.SemaphoreType.DMA,
        pltpu.SemaphoreType.DMA,
        pltpu.SemaphoreType.DMA,
        pltpu.SemaphoreType.DMA,
    ],
)
def _sc_counts(kdst_hbm, out_hbm, kd_all, kd3, ones_v, cbuf, caccum,
               cs0, cs1, cs2, cs3, cs4):
    c = lax.axis_index("c")
    s = lax.axis_index("s")
    sems = (cs0, cs1, cs2, cs3, cs4)
    zro = jnp.zeros((16,), jnp.float32)

    def zrow(i, _):
        cbuf[pl.ds(i * 16, 16)] = zro
        return 0
    lax.fori_loop(0, SPT // 16, zrow, 0)
    for g in range(CH // 16):
        ones_v[pl.ds(g * 16, 16)] = jnp.ones((16,), jnp.float32)
    pltpu.sync_copy(cbuf, caccum.at[pl.ds(s * SPT, SPT)])
    base0 = (c * 16 + s) * EPT
    pltpu.sync_copy(kdst_hbm.at[pl.ds(base0, EPT)], kd_all)

    def reidx(t, _):
        for g in range(CH // 16):
            kd3[t, 0, pl.ds(g * 16, 16)] = kd_all[pl.ds(t * CH + g * 16, 16)]
        return 0
    lax.fori_loop(0, NCH, reidx, 0)
    plsc.subcore_barrier()

    def c_issue(t, k):
        pltpu.async_copy(ones_v, caccum.at[kd3.at[t, 0]], sems[k], add=True)

    def c_drain(k):
        pltpu.make_async_copy(ones_v, caccum.at[kd3.at[0, 0]], sems[k]).wait()

    for k in range(CRING):
        c_issue(k, k)

    def loop(i, _):
        for k in range(CRING):
            c_drain(k)
            c_issue(i * CRING + k, k)
        return 0
    lax.fori_loop(1, NCH // CRING, loop, 0)
    for k in range(CRING):
        c_drain(k)
    plsc.subcore_barrier()
    pltpu.sync_copy(caccum.at[pl.ds(s * SPT, SPT)], cbuf)
    pltpu.sync_copy(cbuf, out_hbm.at[pl.ds(c * NSEG + s * SPT, SPT)])


@functools.partial(
    pl.kernel,
    out_type=jax.ShapeDtypeStruct((2, NPAD, D), jnp.float32),
    mesh=_MESH,
    scratch_types=[
        pltpu.VMEM((CHE,), jnp.int32),
        pltpu.VMEM((CHE,), jnp.int32),
        pltpu.VMEM((CHE,), jnp.int32),
        pltpu.VMEM((CHE,), jnp.int32),
        pltpu.VMEM((CHE,), jnp.int32),
        pltpu.VMEM((CHE,), jnp.int32),
        pltpu.VMEM((CHE,), jnp.float32),
        pltpu.VMEM((CHE,), jnp.float32),
        pltpu.VMEM((CHE, D), jnp.float32),
        pltpu.VMEM((CHE, D), jnp.float32),
        pltpu.VMEM((CHE, D), jnp.float32),
        pltpu.VMEM((CHE, D), jnp.float32),
        pltpu.VMEM((1024,), jnp.float32),
        pltpu.VMEM_SHARED((NSEGP,), jnp.float32),
        pltpu.VMEM_SHARED((NPAD, D), jnp.float32),
        pltpu.SemaphoreType.DMA,
        pltpu.SemaphoreType.DMA,
        pltpu.SemaphoreType.DMA,
        pltpu.SemaphoreType.DMA,
        pltpu.SemaphoreType.DMA,
        pltpu.SemaphoreType.DMA,
        pltpu.SemaphoreType.DMA,
        pltpu.SemaphoreType.DMA,
    ],
)
def _sc_edges(ksrc_hbm, kdst_hbm, xt_hbm, invw_hbm, out_hbm,
              ks0, ks1, kd0, kd1, dt0, dt1, w0, w1, rows0, rows1, sb0, sb1,
              tstage, tblspm, accum, is0, is1, gr0, gr1, gw0, gw1,
              ss0, ss1):
    c = lax.axis_index("c")
    s = lax.axis_index("s")
    zro = jnp.zeros((16,), jnp.float32)
    ksb = (ks0, ks1)
    kdb = (kd0, kd1)
    dtb = (dt0, dt1)
    wb = (w0, w1)
    rb = (rows0, rows1)
    sbf = (sb0, sb1)
    isem = (is0, is1)
    grs = (gr0, gr1)
    gws = (gw0, gw1)
    sss = (ss0, ss1)

    def zrow(i, _):
        for j in range(D // 16):
            sb0[i, pl.ds(j * 16, 16)] = zro
        return 0
    lax.fori_loop(0, CHE, zrow, 0)
    for k in range(RPT // CHE):
        pltpu.sync_copy(sb0, accum.at[pl.ds(s * RPT + k * CHE, CHE)])
    for k in range(SPTP // 1024):
        pltpu.sync_copy(invw_hbm.at[pl.ds(s * SPTP + k * 1024, 1024)], tstage)
        pltpu.sync_copy(tstage, tblspm.at[pl.ds(s * SPTP + k * 1024, 1024)])
    plsc.subcore_barrier()

    base0 = (c * 16 + s) * EPTP

    def i_start(t, b):
        pltpu.async_copy(ksrc_hbm.at[pl.ds(base0 + t * CHE, CHE)],
                         ksb[b], isem[b])
        pltpu.async_copy(kdst_hbm.at[pl.ds(base0 + t * CHE, CHE)],
                         kdb[b], isem[b])

    def i_wait(b):
        pltpu.make_async_copy(ksrc_hbm.at[pl.ds(0, CHE)], ksb[b],
                              isem[b]).wait()
        pltpu.make_async_copy(kdst_hbm.at[pl.ds(0, CHE)], kdb[b],
                              isem[b]).wait()

    def g_start(b):
        pltpu.async_copy(xt_hbm.at[ksb[b]], rb[b], grs[b])
        pltpu.async_copy(tblspm.at[kdb[b]], wb[b], gws[b])

    def g_wait(b):
        pltpu.make_async_copy(xt_hbm.at[ksb[b]], rb[b], grs[b]).wait()
        pltpu.make_async_copy(tblspm.at[kdb[b]], wb[b], gws[b]).wait()

    def s_start(b):
        pltpu.async_copy(sbf[b], accum.at[dtb[b]], sss[b], add=True)

    def s_wait(b):
        pltpu.make_async_copy(sbf[b], accum.at[dtb[b]], sss[b]).wait()

    def dstcalc(b):
        for g in range(CHE // 16):
            dtb[b][pl.ds(g * 16, 16)] = lax.shift_right_logical(
                kdb[b][pl.ds(g * 16, 16)], 4)

    def rowmul(b):
        rv, sv, wv = rb[b], sbf[b], wb[b]

        @plsc.parallel_loop(0, CHE // 16, 1, unroll=2)
        def grp(g):
            w16 = wv[pl.ds(g * 16, 16)]
            for l in range(16):
                wi = w16[l]
                row = g * 16 + l
                for j in range(D // 16):
                    sv[row, pl.ds(j * 16, 16)] = (
                        rv[row, pl.ds(j * 16, 16)] * wi)

    def sub(t, b, head=False, more_idx=True, last=False):
        b1 = 1 - b
        if not head:
            s_wait(b)
        if not last:
            i_wait(b1)
            g_start(b1)
        g_wait(b)
        dstcalc(b)
        if more_idx:
            i_start(t + 2, b)
        rowmul(b)
        s_start(b)

    i_start(0, 0)
    i_wait(0)
    g_start(0)
    i_start(1, 1)
    sub(0, 0, head=True)
    sub(1, 1, head=True)

    def steady(i, _):
        sub(2 * i, 0)
        sub(2 * i + 1, 1)
        return 0
    lax.fori_loop(1, NCHE // 2 - 1, steady, 0)

    sub(NCHE - 2, 0, more_idx=False)
    sub(NCHE - 1, 1, more_idx=False, last=True)
    s_wait(0)
    s_wait(1)
    plsc.subcore_barrier()
    for k in range(RPT // CHE):
        pltpu.sync_copy(accum.at[pl.ds(s * RPT + k * CHE, CHE)], sb0)
        pltpu.sync_copy(sb0, out_hbm.at[c, pl.ds(s * RPT + k * CHE, CHE)])


def kernel(adj_t, edge_types, emb, basis1, comp1, root1, bias1,
           basis2, comp2, root2, bias2):
    src = adj_t[0]
    dst = adj_t[1]
    ksrc, kdst = _make_keys(src, dst, edge_types)

    wrel1 = _make_wrel(basis1, comp1)
    wrel2 = _make_wrel(basis2, comp2)

    cparts = _sc_counts(kdst).reshape(2, NSEG)
    invw = _make_invw(cparts)

    pad = jnp.arange(EPAD - E, dtype=jnp.int32)
    ksrc = jnp.concatenate([ksrc, pad % NSEG])
    kdst = jnp.concatenate([kdst, NSEG + (pad % 240) * 16])
    invw = jnp.concatenate([invw, jnp.zeros((NSEGP - NSEG,), jnp.float32)])

    xt1 = _matmul(emb, wrel1)
    sc1 = _sc_edges(ksrc, kdst, xt1, invw)
    x1 = _combine(sc1[0, :N], sc1[1, :N], emb, root1, bias1)

    xt2 = _matmul(x1, wrel2)
    sc2 = _sc_edges(ksrc, kdst, xt2, invw)
    return _final(sc2[0, :N], sc2[1, :N], x1, emb, root2, bias2)


# speedup vs baseline: 1.0181x; 1.0181x over previous
import functools

import jax
import jax.numpy as jnp
from jax import lax
from jax.experimental import pallas as pl
from jax.experimental.pallas import tpu as pltpu
from jax.experimental.pallas import tpu_sc as plsc

N = 10000
R = 16
NB = 12
D = 128
E = 320000
NSEG = N * R
NTILES = 32
EPT = E // NTILES
CH = 80
NCH = EPT // CH
NPAD = 10240
RPT = NPAD // 16
BROWS = 64
SPT = NSEG // 16
EPAD = 327680
EPTP = EPAD // NTILES
CHE = 32
NCHE = EPTP // CHE
NSEGP = NPAD * R
SPTP = NSEGP // 16


def _keys_body(src_ref, dst_ref, ty_ref, ksrc_ref, kdst_ref):
    ksrc_ref[...] = ty_ref[...] * N + src_ref[...]
    kdst_ref[...] = dst_ref[...] * R + ty_ref[...]


def _make_keys(src, dst, ty):
    s2 = src.reshape(2500, 128)
    d2 = dst.reshape(2500, 128)
    t2 = ty.reshape(2500, 128)
    ks, kd = pl.pallas_call(
        _keys_body,
        out_shape=(jax.ShapeDtypeStruct((2500, 128), jnp.int32),
                   jax.ShapeDtypeStruct((2500, 128), jnp.int32)),
    )(s2, d2, t2)
    return ks.reshape(E), kd.reshape(E)


def _wrel_body(comp_ref, basis_ref, out_ref):
    r = pl.program_id(0)
    acc = comp_ref[r, 0] * basis_ref[0]
    for b in range(1, NB):
        acc = acc + comp_ref[r, b] * basis_ref[b]
    out_ref[...] = acc


def _make_wrel(basis, comp):
    wrel = pl.pallas_call(
        _wrel_body,
        grid=(R,),
        in_specs=[
            pl.BlockSpec(memory_space=pltpu.SMEM),
            pl.BlockSpec((NB, D, D), lambda r: (0, 0, 0)),
        ],
        out_specs=pl.BlockSpec((D, D), lambda r: (0, r)),
        out_shape=jax.ShapeDtypeStruct((D, R * D), jnp.float32),
    )(comp, basis)
    return wrel


def _mm_body(x_ref, w_ref, xt_ref):
    xt_ref[...] = jnp.dot(x_ref[...], w_ref[...],
                          preferred_element_type=jnp.float32)


def _matmul(x, wrel):
    return pl.pallas_call(
        _mm_body,
        grid=(R, 10),
        in_specs=[
            pl.BlockSpec((1000, D), lambda r, i: (i, 0)),
            pl.BlockSpec((D, D), lambda r, i: (0, r)),
        ],
        out_specs=pl.BlockSpec((1000, D), lambda r, i: (r * 10 + i, 0)),
        out_shape=jax.ShapeDtypeStruct((NSEG, D), jnp.float32),
    )(x, wrel)


def _inv_body(c0_ref, c1_ref, out_ref):
    c = c0_ref[...] + c1_ref[...]
    out_ref[...] = 1.0 / jnp.maximum(c, 1.0)


def _make_invw(cparts):
    c0 = cparts[0].reshape(1250, 128)
    c1 = cparts[1].reshape(1250, 128)
    inv = pl.pallas_call(
        _inv_body,
        out_shape=jax.ShapeDtypeStruct((1250, 128), jnp.float32),
    )(c0, c1)
    return inv.reshape(NSEG)


def _comb_body(p0_ref, p1_ref, x_ref, root_ref, b_ref, out_ref):
    rp = jnp.dot(x_ref[...], root_ref[...], preferred_element_type=jnp.float32)
    s = p0_ref[0] + p1_ref[0] + rp + b_ref[...]
    out_ref[...] = jnp.maximum(s, 0.0)


def _combine(scp, x, root, bias):
    return pl.pallas_call(
        _comb_body,
        grid=(10,),
        in_specs=[
            pl.BlockSpec((1, 1000, D), lambda i: (0, i, 0)),
            pl.BlockSpec((1, 1000, D), lambda i: (1, i, 0)),
            pl.BlockSpec((1000, D), lambda i: (i, 0)),
            pl.BlockSpec((D, D), lambda i: (0, 0)),
            pl.BlockSpec((1, D), lambda i: (0, 0)),
        ],
        out_specs=pl.BlockSpec((1000, D), lambda i: (i, 0)),
        out_shape=jax.ShapeDtypeStruct((N, D), jnp.float32),
    )(scp, scp, x, root, bias.reshape(1, D))


def _final_body(p0_ref, p1_ref, x1_ref, emb_ref, root_ref, b_ref, out_ref):
    rp = jnp.dot(x1_ref[...], root_ref[...], preferred_element_type=jnp.float32)
    x2 = jnp.maximum(p0_ref[0] + p1_ref[0] + rp + b_ref[...], 0.0)
    out_ref[...] = jnp.concatenate((x2, x1_ref[...], emb_ref[...]), axis=1)


def _final(scp, x1, emb, root, bias):
    return pl.pallas_call(
        _final_body,
        grid=(10,),
        in_specs=[
            pl.BlockSpec((1, 1000, D), lambda i: (0, i, 0)),
            pl.BlockSpec((1, 1000, D), lambda i: (1, i, 0)),
            pl.BlockSpec((1000, D), lambda i: (i, 0)),
            pl.BlockSpec((1000, D), lambda i: (i, 0)),
            pl.BlockSpec((D, D), lambda i: (0, 0)),
            pl.BlockSpec((1, D), lambda i: (0, 0)),
        ],
        out_specs=pl.BlockSpec((1000, 3 * D), lambda i: (i, 0)),
        out_shape=jax.ShapeDtypeStruct((N, 3 * D), jnp.float32),
    )(scp, scp, x1, emb, root, bias.reshape(1, D))


_MESH = plsc.VectorSubcoreMesh(core_axis_name="c", subcore_axis_name="s")


CRING = 5


@functools.partial(
    pl.kernel,
    out_type=jax.ShapeDtypeStruct((2 * NSEG,), jnp.float32),
    mesh=_MESH,
    scratch_types=[
        pltpu.VMEM((EPT,), jnp.int32),
        pltpu.VMEM((NCH, 1, CH), jnp.int32),
        pltpu.VMEM((CH,), jnp.float32),
        pltpu.VMEM((SPT,), jnp.float32),
        pltpu.VMEM_SHARED((NSEG,), jnp.float32),
        pltpu.SemaphoreType.DMA,
        pltpu.SemaphoreType.DMA,
        pltpu.SemaphoreType.DMA,
        pltpu.SemaphoreType.DMA,
        pltpu.SemaphoreType.DMA,
    ],
)
def _sc_counts(kdst_hbm, out_hbm, kd_all, kd3, ones_v, cbuf, caccum,
               cs0, cs1, cs2, cs3, cs4):
    c = lax.axis_index("c")
    s = lax.axis_index("s")
    sems = (cs0, cs1, cs2, cs3, cs4)
    zro = jnp.zeros((16,), jnp.float32)

    def zrow(i, _):
        cbuf[pl.ds(i * 16, 16)] = zro
        return 0
    lax.fori_loop(0, SPT // 16, zrow, 0)
    for g in range(CH // 16):
        ones_v[pl.ds(g * 16, 16)] = jnp.ones((16,), jnp.float32)
    pltpu.sync_copy(cbuf, caccum.at[pl.ds(s * SPT, SPT)])
    base0 = (c * 16 + s) * EPT
    pltpu.sync_copy(kdst_hbm.at[pl.ds(base0, EPT)], kd_all)

    def reidx(t, _):
        for g in range(CH // 16):
            kd3[t, 0, pl.ds(g * 16, 16)] = kd_all[pl.ds(t * CH + g * 16, 16)]
        return 0
    lax.fori_loop(0, NCH, reidx, 0)
    plsc.subcore_barrier()

    def c_issue(t, k):
        pltpu.async_copy(ones_v, caccum.at[kd3.at[t, 0]], sems[k], add=True)

    def c_drain(k):
        pltpu.make_async_copy(ones_v, caccum.at[kd3.at[0, 0]], sems[k]).wait()

    for k in range(CRING):
        c_issue(k, k)

    def loop(i, _):
        for k in range(CRING):
            c_drain(k)
            c_issue(i * CRING + k, k)
        return 0
    lax.fori_loop(1, NCH // CRING, loop, 0)
    for k in range(CRING):
        c_drain(k)
    plsc.subcore_barrier()
    pltpu.sync_copy(caccum.at[pl.ds(s * SPT, SPT)], cbuf)
    pltpu.sync_copy(cbuf, out_hbm.at[pl.ds(c * NSEG + s * SPT, SPT)])


@functools.partial(
    pl.kernel,
    out_type=jax.ShapeDtypeStruct((2, NPAD, D), jnp.float32),
    mesh=_MESH,
    scratch_types=[
        pltpu.VMEM((CHE,), jnp.int32),
        pltpu.VMEM((CHE,), jnp.int32),
        pltpu.VMEM((CHE,), jnp.int32),
        pltpu.VMEM((CHE,), jnp.int32),
        pltpu.VMEM((CHE,), jnp.int32),
        pltpu.VMEM((CHE,), jnp.int32),
        pltpu.VMEM((CHE,), jnp.float32),
        pltpu.VMEM((CHE,), jnp.float32),
        pltpu.VMEM((CHE, D), jnp.float32),
        pltpu.VMEM((CHE, D), jnp.float32),
        pltpu.VMEM((CHE, D), jnp.float32),
        pltpu.VMEM((CHE, D), jnp.float32),
        pltpu.VMEM((1024,), jnp.float32),
        pltpu.VMEM_SHARED((NSEGP,), jnp.float32),
        pltpu.VMEM_SHARED((NPAD, D), jnp.float32),
        pltpu.SemaphoreType.DMA,
        pltpu.SemaphoreType.DMA,
        pltpu.SemaphoreType.DMA,
        pltpu.SemaphoreType.DMA,
        pltpu.SemaphoreType.DMA,
        pltpu.SemaphoreType.DMA,
        pltpu.SemaphoreType.DMA,
        pltpu.SemaphoreType.DMA,
    ],
)
def _sc_edges(ksrc_hbm, kdst_hbm, xt_hbm, invw_hbm, out_hbm,
              ks0, ks1, kd0, kd1, dt0, dt1, w0, w1, rows0, rows1, sb0, sb1,
              tstage, tblspm, accum, is0, is1, gr0, gr1, gw0, gw1,
              ss0, ss1):
    c = lax.axis_index("c")
    s = lax.axis_index("s")
    zro = jnp.zeros((16,), jnp.float32)
    ksb = (ks0, ks1)
    kdb = (kd0, kd1)
    dtb = (dt0, dt1)
    wb = (w0, w1)
    rb = (rows0, rows1)
    sbf = (sb0, sb1)
    isem = (is0, is1)
    grs = (gr0, gr1)
    gws = (gw0, gw1)
    sss = (ss0, ss1)

    def zrow(i, _):
        for j in range(D // 16):
            sb0[i, pl.ds(j * 16, 16)] = zro
        return 0
    lax.fori_loop(0, CHE, zrow, 0)
    for k in range(RPT // CHE):
        pltpu.sync_copy(sb0, accum.at[pl.ds(s * RPT + k * CHE, CHE)])
    for k in range(SPTP // 1024):
        pltpu.sync_copy(invw_hbm.at[pl.ds(s * SPTP + k * 1024, 1024)], tstage)
        pltpu.sync_copy(tstage, tblspm.at[pl.ds(s * SPTP + k * 1024, 1024)])
    plsc.subcore_barrier()

    base0 = (c * 16 + s) * EPTP

    def i_start(t, b):
        pltpu.async_copy(ksrc_hbm.at[pl.ds(base0 + t * CHE, CHE)],
                         ksb[b], isem[b])
        pltpu.async_copy(kdst_hbm.at[pl.ds(base0 + t * CHE, CHE)],
                         kdb[b], isem[b])

    def i_wait(b):
        pltpu.make_async_copy(ksrc_hbm.at[pl.ds(0, CHE)], ksb[b],
                              isem[b]).wait()
        pltpu.make_async_copy(kdst_hbm.at[pl.ds(0, CHE)], kdb[b],
                              isem[b]).wait()

    def g_start(b):
        pltpu.async_copy(xt_hbm.at[ksb[b]], rb[b], grs[b])
        pltpu.async_copy(tblspm.at[kdb[b]], wb[b], gws[b])

    def g_wait(b):
        pltpu.make_async_copy(xt_hbm.at[ksb[b]], rb[b], grs[b]).wait()
        pltpu.make_async_copy(tblspm.at[kdb[b]], wb[b], gws[b]).wait()

    def s_start(b):
        pltpu.async_copy(sbf[b], accum.at[dtb[b]], sss[b], add=True)

    def s_wait(b):
        pltpu.make_async_copy(sbf[b], accum.at[dtb[b]], sss[b]).wait()

    def dstcalc(b):
        for g in range(CHE // 16):
            dtb[b][pl.ds(g * 16, 16)] = lax.shift_right_logical(
                kdb[b][pl.ds(g * 16, 16)], 4)

    def rowmul(b):
        rv, sv, wv = rb[b], sbf[b], wb[b]

        @plsc.parallel_loop(0, CHE // 16, 1, unroll=2)
        def grp(g):
            w16 = wv[pl.ds(g * 16, 16)]
            for l in range(16):
                wi = w16[l]
                row = g * 16 + l
                for j in range(D // 16):
                    sv[row, pl.ds(j * 16, 16)] = (
                        rv[row, pl.ds(j * 16, 16)] * wi)

    def sub(t, b, head=False, more_idx=True, last=False):
        b1 = 1 - b
        if not head:
            s_wait(b)
        if not last:
            i_wait(b1)
            g_start(b1)
        g_wait(b)
        dstcalc(b)
        if more_idx:
            i_start(t + 2, b)
        rowmul(b)
        s_start(b)

    i_start(0, 0)
    i_wait(0)
    g_start(0)
    i_start(1, 1)
    sub(0, 0, head=True)
    sub(1, 1, head=True)

    def steady(i, _):
        sub(2 * i, 0)
        sub(2 * i + 1, 1)
        return 0
    lax.fori_loop(1, NCHE // 2 - 1, steady, 0)

    sub(NCHE - 2, 0, more_idx=False)
    sub(NCHE - 1, 1, more_idx=False, last=True)
    s_wait(0)
    s_wait(1)
    plsc.subcore_barrier()
    for k in range(RPT // CHE):
        pltpu.sync_copy(accum.at[pl.ds(s * RPT + k * CHE, CHE)], sb0)
        pltpu.sync_copy(sb0, out_hbm.at[c, pl.ds(s * RPT + k * CHE, CHE)])


def kernel(adj_t, edge_types, emb, basis1, comp1, root1, bias1,
           basis2, comp2, root2, bias2):
    src = adj_t[0]
    dst = adj_t[1]
    ksrc, kdst = _make_keys(src, dst, edge_types)

    wrel1 = _make_wrel(basis1, comp1)
    wrel2 = _make_wrel(basis2, comp2)

    cparts = _sc_counts(kdst).reshape(2, NSEG)
    invw = _make_invw(cparts)

    pad = jnp.arange(EPAD - E, dtype=jnp.int32)
    ksrc = jnp.concatenate([ksrc, pad % NSEG])
    kdst = jnp.concatenate([kdst, NSEG + (pad % 240) * 16])
    invw = jnp.concatenate([invw, jnp.zeros((NSEGP - NSEG,), jnp.float32)])

    xt1 = _matmul(emb, wrel1)
    sc1 = _sc_edges(ksrc, kdst, xt1, invw)
    x1 = _combine(sc1, emb, root1, bias1)

    xt2 = _matmul(x1, wrel2)
    sc2 = _sc_edges(ksrc, kdst, xt2, invw)
    return _final(sc2, x1, emb, root2, bias2)
